# baseline (speedup 1.0000x reference)
import functools

import jax
import jax.numpy as jnp
from jax import lax
from jax.experimental import pallas as pl
from jax.experimental.pallas import tpu as pltpu
from jax.experimental.pallas import tpu_sc as plsc

B, S, F = 32, 2048, 512
SEG = 256
LANES = 16
WIN = SEG + 8
CHG = 88


def _gather_sc(x, indices, starts):
    mesh = plsc.VectorSubcoreMesh(core_axis_name="c", subcore_axis_name="s")
    info = plsc.get_sparse_core_info()
    nc = info.num_cores

    @functools.partial(
        pl.kernel,
        mesh=mesh,
        out_type=jax.ShapeDtypeStruct((B, WIN, F), jnp.float32),
        scratch_types=(
            [pltpu.VMEM((B + 16,), jnp.int32)] * 2
            + [pltpu.VMEM((CHG, F), jnp.float32)] * 2
            + [pltpu.VMEM((8, F), jnp.float32)] * 2
            + [pltpu.SemaphoreType.DMA] * 8
        ),
    )
    def k(x_hbm, idx_hbm, st_hbm, seg_hbm, idx_v, st_v,
          buf0, buf1, eb0, eb2, g0, g1, g2, e0, e2, s0, s1, s2):
        wid = lax.axis_index("s") * nc + lax.axis_index("c")
        pltpu.sync_copy(idx_hbm, idx_v.at[pl.ds(0, B)])
        pltpu.sync_copy(st_hbm, st_v.at[pl.ds(0, B)])
        p = idx_v[pl.ds(wid, LANES)][0]
        st = st_v[pl.ds(wid, LANES)][0]
        m = lax.rem(st, 8)
        a0 = pl.multiple_of(st - m, 8)

        hg0 = pltpu.async_copy(x_hbm.at[p, pl.ds(a0, CHG)], buf0, g0)
        hg1 = pltpu.async_copy(x_hbm.at[p, pl.ds(a0 + CHG, CHG)], buf1, g1)
        he0 = pltpu.async_copy(x_hbm.at[wid, pl.ds(a0, 8)], eb0, e0)
        he2 = pltpu.async_copy(x_hbm.at[wid, pl.ds(a0 + SEG, 8)], eb2, e2)

        hg0.wait()
        he0.wait()
        for r in range(8):
            @pl.when(r < m)
            def _lead():
                for c in range(F // LANES):
                    sl = pl.ds(c * LANES, LANES)
                    buf0[r, sl] = eb0[r, sl]
        hs0 = pltpu.async_copy(buf0, seg_hbm.at[wid, pl.ds(0, CHG)], s0)

        hg1.wait()
        hs1 = pltpu.async_copy(buf1, seg_hbm.at[wid, pl.ds(CHG, CHG)], s1)

        hs0.wait()
        hg2 = pltpu.async_copy(x_hbm.at[p, pl.ds(a0 + 2 * CHG, CHG)], buf0, g2)
        hg2.wait()
        he2.wait()
        for r in range(8):
            @pl.when(r >= m)
            def _trail():
                for c in range(F // LANES):
                    sl = pl.ds(c * LANES, LANES)
                    buf0[80 + r, sl] = eb2[r, sl]
        hs2 = pltpu.async_copy(buf0, seg_hbm.at[wid, pl.ds(2 * CHG, CHG)], s2)

        hs1.wait()
        hs2.wait()

    return k(x, indices, starts)


def _combine_tc(x, seg, starts):
    def body(st_ref, x_ref, seg_ref, out_ref, row_sem, patch_sem):
        hs = []
        for i in range(B):
            h = pltpu.make_async_copy(x_ref.at[i], out_ref.at[i], row_sem.at[i])
            h.start()
            hs.append(h)
        ps = []
        for i in range(B):
            hs[i].wait()
            st = st_ref[i]
            a0 = pl.multiple_of(st - lax.rem(st, 8), 8)
            hp = pltpu.make_async_copy(
                seg_ref.at[i], out_ref.at[i, pl.ds(a0, WIN)], patch_sem.at[i])
            hp.start()
            ps.append(hp)
        for hp in ps:
            hp.wait()

    return pl.pallas_call(
        body,
        out_shape=jax.ShapeDtypeStruct((B, S, F), jnp.float32),
        in_specs=[
            pl.BlockSpec(memory_space=pltpu.SMEM),
            pl.BlockSpec(memory_space=pl.ANY),
            pl.BlockSpec(memory_space=pl.ANY),
        ],
        out_specs=pl.BlockSpec(memory_space=pl.ANY),
        scratch_shapes=[
            pltpu.SemaphoreType.DMA((B,)),
            pltpu.SemaphoreType.DMA((B,)),
        ],
    )(starts, x, seg)


def kernel(x, indices, starts):
    seg = _gather_sc(x, indices, starts)
    return _combine_tc(x, seg, starts)


# speedup vs baseline: 37.0147x; 37.0147x over previous
import functools

import jax
import jax.numpy as jnp
from jax import lax
from jax.experimental import pallas as pl
from jax.experimental.pallas import tpu as pltpu
from jax.experimental.pallas import tpu_sc as plsc

B, S, F = 32, 2048, 512
SEG = 256
LANES = 16
WIN = SEG + 8
CHG = 88


def _gather_sc(x, indices, starts):
    mesh = plsc.VectorSubcoreMesh(core_axis_name="c", subcore_axis_name="s")
    info = plsc.get_sparse_core_info()
    nc = info.num_cores

    @functools.partial(
        pl.kernel,
        mesh=mesh,
        out_type=jax.ShapeDtypeStruct((B, WIN, F), jnp.float32),
        scratch_types=(
            [pltpu.VMEM((B + 16,), jnp.int32)] * 2
            + [pltpu.VMEM((CHG, F), jnp.float32)] * 2
            + [pltpu.VMEM((8, F), jnp.float32)] * 2
            + [pltpu.SemaphoreType.DMA] * 8
        ),
    )
    def k(x_hbm, idx_hbm, st_hbm, seg_hbm, idx_v, st_v,
          buf0, buf1, eb0, eb2, g0, g1, g2, e0, e2, s0, s1, s2):
        wid = lax.axis_index("s") * nc + lax.axis_index("c")
        pltpu.sync_copy(idx_hbm, idx_v.at[pl.ds(0, B)])
        pltpu.sync_copy(st_hbm, st_v.at[pl.ds(0, B)])
        p = idx_v[pl.ds(wid, LANES)][0]
        st = st_v[pl.ds(wid, LANES)][0]
        m = lax.rem(st, 8)
        a0 = pl.multiple_of(st - m, 8)

        hg0 = pltpu.async_copy(x_hbm.at[p, pl.ds(a0, CHG)], buf0, g0)
        hg1 = pltpu.async_copy(x_hbm.at[p, pl.ds(a0 + CHG, CHG)], buf1, g1)
        he0 = pltpu.async_copy(x_hbm.at[wid, pl.ds(a0, 8)], eb0, e0)
        he2 = pltpu.async_copy(x_hbm.at[wid, pl.ds(a0 + SEG, 8)], eb2, e2)

        hg0.wait()
        he0.wait()
        for r in range(8):
            @pl.when(r < m)
            def _lead():
                for c in range(F // LANES):
                    sl = pl.ds(c * LANES, LANES)
                    buf0[r, sl] = eb0[r, sl]
        hs0 = pltpu.async_copy(buf0, seg_hbm.at[wid, pl.ds(0, CHG)], s0)

        hg1.wait()
        hs1 = pltpu.async_copy(buf1, seg_hbm.at[wid, pl.ds(CHG, CHG)], s1)

        hs0.wait()
        hg2 = pltpu.async_copy(x_hbm.at[p, pl.ds(a0 + 2 * CHG, CHG)], buf0, g2)
        hg2.wait()
        he2.wait()
        for r in range(8):
            @pl.when(r >= m)
            def _trail():
                for c in range(F // LANES):
                    sl = pl.ds(c * LANES, LANES)
                    buf0[80 + r, sl] = eb2[r, sl]
        hs2 = pltpu.async_copy(buf0, seg_hbm.at[wid, pl.ds(2 * CHG, CHG)], s2)

        hs1.wait()
        hs2.wait()

    return k(x, indices, starts)


NB = 4


def _combine_tc(x, seg, starts):
    def body(st_ref, x_ref, seg_ref, out_ref, *scr):
        bufs = list(scr[:NB])
        in_sem, seg_sem, out_sem = scr[NB:]

        in_h = [None] * B
        seg_h = [None] * B
        out_h = [None] * B

        def start_in(i):
            b = i % NB
            in_h[i] = pltpu.make_async_copy(x_ref.at[i], bufs[b], in_sem.at[b])
            in_h[i].start()

        def start_seg(i):
            b = i % NB
            st = st_ref[i]
            a0 = pl.multiple_of(st - lax.rem(st, 8), 8)
            seg_h[i] = pltpu.make_async_copy(
                seg_ref.at[i], bufs[b].at[pl.ds(a0, WIN)], seg_sem.at[b])
            seg_h[i].start()

        def start_out(i):
            b = i % NB
            out_h[i] = pltpu.make_async_copy(bufs[b], out_ref.at[i],
                                             out_sem.at[b])
            out_h[i].start()

        for t in range(B + 2):
            if t < B:
                if t >= NB:
                    out_h[t - NB].wait()
                start_in(t)
            if 1 <= t <= B:
                in_h[t - 1].wait()
                start_seg(t - 1)
            if 2 <= t <= B + 1:
                seg_h[t - 2].wait()
                start_out(t - 2)
        for i in range(B - NB, B):
            out_h[i].wait()

    return pl.pallas_call(
        body,
        out_shape=jax.ShapeDtypeStruct((B, S, F), jnp.float32),
        in_specs=[
            pl.BlockSpec(memory_space=pltpu.SMEM),
            pl.BlockSpec(memory_space=pl.ANY),
            pl.BlockSpec(memory_space=pl.ANY),
        ],
        out_specs=pl.BlockSpec(memory_space=pl.ANY),
        scratch_shapes=(
            [pltpu.VMEM((S, F), jnp.float32)] * NB
            + [pltpu.SemaphoreType.DMA((NB,))] * 3
        ),
    )(starts, x, seg)


def kernel(x, indices, starts):
    seg = _gather_sc(x, indices, starts)
    return _combine_tc(x, seg, starts)


# speedup vs baseline: 37.0980x; 1.0022x over previous
import functools

import jax
import jax.numpy as jnp
from jax import lax
from jax.experimental import pallas as pl
from jax.experimental.pallas import tpu as pltpu
from jax.experimental.pallas import tpu_sc as plsc

B, S, F = 32, 2048, 512
SEG = 256
LANES = 16
WIN = SEG + 8
CHG = 88


def _gather_sc(x, indices, starts):
    mesh = plsc.VectorSubcoreMesh(core_axis_name="c", subcore_axis_name="s")
    info = plsc.get_sparse_core_info()
    nc = info.num_cores

    @functools.partial(
        pl.kernel,
        mesh=mesh,
        out_type=jax.ShapeDtypeStruct((B, WIN, F), jnp.float32),
        scratch_types=(
            [pltpu.VMEM((B + 16,), jnp.int32)] * 2
            + [pltpu.VMEM((CHG, F), jnp.float32)] * 2
            + [pltpu.VMEM((8, F), jnp.float32)] * 2
            + [pltpu.SemaphoreType.DMA] * 8
        ),
    )
    def k(x_hbm, idx_hbm, st_hbm, seg_hbm, idx_v, st_v,
          buf0, buf1, eb0, eb2, g0, g1, g2, e0, e2, s0, s1, s2):
        wid = lax.axis_index("s") * nc + lax.axis_index("c")
        pltpu.sync_copy(idx_hbm, idx_v.at[pl.ds(0, B)])
        pltpu.sync_copy(st_hbm, st_v.at[pl.ds(0, B)])
        p = idx_v[pl.ds(wid, LANES)][0]
        st = st_v[pl.ds(wid, LANES)][0]
        m = lax.rem(st, 8)
        a0 = pl.multiple_of(st - m, 8)

        hg0 = pltpu.async_copy(x_hbm.at[p, pl.ds(a0, CHG)], buf0, g0)
        hg1 = pltpu.async_copy(x_hbm.at[p, pl.ds(a0 + CHG, CHG)], buf1, g1)
        he0 = pltpu.async_copy(x_hbm.at[wid, pl.ds(a0, 8)], eb0, e0)
        he2 = pltpu.async_copy(x_hbm.at[wid, pl.ds(a0 + SEG, 8)], eb2, e2)

        hg0.wait()
        he0.wait()
        for r in range(8):
            @pl.when(r < m)
            def _lead():
                for c in range(F // LANES):
                    sl = pl.ds(c * LANES, LANES)
                    buf0[r, sl] = eb0[r, sl]
        hs0 = pltpu.async_copy(buf0, seg_hbm.at[wid, pl.ds(0, CHG)], s0)

        hg1.wait()
        hs1 = pltpu.async_copy(buf1, seg_hbm.at[wid, pl.ds(CHG, CHG)], s1)

        hs0.wait()
        hg2 = pltpu.async_copy(x_hbm.at[p, pl.ds(a0 + 2 * CHG, CHG)], buf0, g2)
        hg2.wait()
        he2.wait()
        for r in range(8):
            @pl.when(r >= m)
            def _trail():
                for c in range(F // LANES):
                    sl = pl.ds(c * LANES, LANES)
                    buf0[80 + r, sl] = eb2[r, sl]
        hs2 = pltpu.async_copy(buf0, seg_hbm.at[wid, pl.ds(2 * CHG, CHG)], s2)

        hs1.wait()
        hs2.wait()

    return k(x, indices, starts)


NB = 8
AHEAD = 4


def _combine_tc(x, seg, starts):
    def body(st_ref, x_ref, seg_ref, out_ref, *scr):
        bufs = list(scr[:NB])
        in_sem, seg_sem, out_sem = scr[NB:]

        in_h = [None] * B
        seg_h = [None] * B
        out_h = [None] * B

        def start_in(i):
            b = i % NB
            in_h[i] = pltpu.make_async_copy(x_ref.at[i], bufs[b], in_sem.at[b])
            in_h[i].start()

        def start_seg(i):
            b = i % NB
            st = st_ref[i]
            a0 = pl.multiple_of(st - lax.rem(st, 8), 8)
            seg_h[i] = pltpu.make_async_copy(
                seg_ref.at[i], bufs[b].at[pl.ds(a0, WIN)], seg_sem.at[b])
            seg_h[i].start()

        def start_out(i):
            b = i % NB
            out_h[i] = pltpu.make_async_copy(bufs[b], out_ref.at[i],
                                             out_sem.at[b])
            out_h[i].start()

        waited = [False] * B
        for t in range(B + AHEAD + 1):
            if t < B:
                if t >= NB:
                    out_h[t - NB].wait()
                    waited[t - NB] = True
                start_in(t)
            u = t - AHEAD
            if 0 <= u < B:
                in_h[u].wait()
                start_seg(u)
            v = t - AHEAD - 1
            if 0 <= v < B:
                seg_h[v].wait()
                start_out(v)
        for i in range(B):
            if not waited[i]:
                out_h[i].wait()

    return pl.pallas_call(
        body,
        out_shape=jax.ShapeDtypeStruct((B, S, F), jnp.float32),
        in_specs=[
            pl.BlockSpec(memory_space=pltpu.SMEM),
            pl.BlockSpec(memory_space=pl.ANY),
            pl.BlockSpec(memory_space=pl.ANY),
        ],
        out_specs=pl.BlockSpec(memory_space=pl.ANY),
        scratch_shapes=(
            [pltpu.VMEM((S, F), jnp.float32)] * NB
            + [pltpu.SemaphoreType.DMA((NB,))] * 3
        ),
    )(starts, x, seg)


def kernel(x, indices, starts):
    seg = _gather_sc(x, indices, starts)
    return _combine_tc(x, seg, starts)
